# baseline (speedup 1.0000x reference)
import functools

import jax
import jax.numpy as jnp
from jax import lax
from jax.experimental import pallas as pl
from jax.experimental.pallas import tpu as pltpu
from jax.experimental.pallas import tpu_sc as plsc

BATCH = 4096
SEQ = 200
HIDDEN = 64
NUM_WORKERS = 32
IBLK = BATCH // NUM_WORKERS
NBUF = 2
NUM_GROUPS = SEQ // NBUF


def _make_kernel():
    mesh = plsc.VectorSubcoreMesh(core_axis_name="c", subcore_axis_name="s")

    @functools.partial(
        pl.kernel,
        out_type=jax.ShapeDtypeStruct((SEQ, 8, 32, 8, 128), jnp.float32),
        mesh=mesh,
        scratch_types=[
            pltpu.VMEM((SEQ, IBLK), jnp.int32),
            pltpu.VMEM((NBUF, IBLK, HIDDEN), jnp.float32),
            pltpu.VMEM((NBUF, 8, 8, 128), jnp.float32),
            [pltpu.SemaphoreType.DMA] * NBUF,
            [pltpu.SemaphoreType.DMA] * NBUF,
        ],
        compiler_params=pltpu.CompilerParams(
            use_tc_tiling_on_sc=False, needs_layout_passes=False
        ),
    )
    def emb_kernel(ids_hbm, table_hbm, out_hbm, idx_all, rows, tbuf, sem_g, sem_w):
        wid = lax.axis_index("s") * 2 + lax.axis_index("c")

        pltpu.sync_copy(ids_hbm.at[:, pl.ds(wid * IBLK, IBLK)], idx_all)

        lane = lax.iota(jnp.int32, 16)
        kvecs = [lane + (kc * 16) for kc in range(HIDDEN // 16)]
        tkvs = [kv // 8 for kv in kvecs]
        k8vs = [kv % 8 for kv in kvecs]

        def gather_start(j, b):
            pltpu.async_copy(
                table_hbm.at[idx_all.at[j]], rows.at[b], sem_g[b]
            )

        def gather_wait(b):
            pltpu.make_async_copy(
                table_hbm.at[idx_all.at[0]], rows.at[b], sem_g[b]
            ).wait()

        def write_start(j, b):
            pltpu.async_copy(tbuf.at[b], out_hbm.at[j, :, wid], sem_w[b])

        def write_wait(b):
            pltpu.make_async_copy(
                tbuf.at[b], out_hbm.at[0, :, wid], sem_w[b]
            ).wait()

        def transpose_block(b):
            rows_b = rows.at[b]
            tbuf_b = tbuf.at[b]

            @plsc.parallel_loop(0, IBLK, unroll=16)
            def i_body(i):
                isplat = jnp.full((16,), i, jnp.int32)
                for kc in range(HIDDEN // 16):
                    v = rows_b[i, pl.ds(kc * 16, 16)]
                    plsc.store_scatter(tbuf_b, [tkvs[kc], k8vs[kc], isplat], v)

        def chunk_body(j, b, issue_gather, wait_write):
            gather_wait(b)
            if wait_write:
                write_wait(b)
            transpose_block(b)
            write_start(j, b)
            if issue_gather:
                gather_start(j + NBUF, b)

        for b in range(NBUF):
            gather_start(b, b)

        for b in range(NBUF):
            chunk_body(b, b, True, False)

        def group_body(g, carry):
            j0 = g * NBUF
            for b in range(NBUF):
                chunk_body(j0 + b, b, True, True)
            return carry

        lax.fori_loop(1, NUM_GROUPS - 1, group_body, 0)

        j0 = (NUM_GROUPS - 1) * NBUF
        for b in range(NBUF):
            chunk_body(j0 + b, b, False, True)

        for b in range(NBUF):
            write_wait(b)

    return emb_kernel


_emb = _make_kernel()


@jax.jit
def kernel(phone_ids, table):
    ids_t = phone_ids.T
    phys = _emb(ids_t, table)
    out = phys.transpose(0, 1, 3, 2, 4).reshape(SEQ, HIDDEN, BATCH)
    return out.transpose(2, 0, 1)


# speedup vs baseline: 1.7897x; 1.7897x over previous
import functools

import jax
import jax.numpy as jnp
from jax import lax
from jax.experimental import pallas as pl
from jax.experimental.pallas import tpu as pltpu
from jax.experimental.pallas import tpu_sc as plsc

BATCH = 4096
SEQ = 200
HIDDEN = 64
NUM_WORKERS = 32
IBLK = BATCH // NUM_WORKERS
NBUF = 2
NUM_GROUPS = SEQ // NBUF


def _make_kernel():
    mesh = plsc.VectorSubcoreMesh(core_axis_name="c", subcore_axis_name="s")

    @functools.partial(
        pl.kernel,
        out_type=jax.ShapeDtypeStruct((SEQ, 8, 32, 8, 128), jnp.float32),
        mesh=mesh,
        scratch_types=[
            pltpu.VMEM((SEQ, IBLK), jnp.int32),
            pltpu.VMEM((NBUF, IBLK, HIDDEN), jnp.float32),
            pltpu.VMEM((NBUF, 8, 8, 128), jnp.float32),
            [pltpu.SemaphoreType.DMA] * NBUF,
            [pltpu.SemaphoreType.DMA] * NBUF,
        ],
        compiler_params=pltpu.CompilerParams(
            use_tc_tiling_on_sc=False, needs_layout_passes=False
        ),
    )
    def emb_kernel(ids_hbm, table_hbm, out_hbm, idx_all, rows, tbuf, sem_g, sem_w):
        wid = lax.axis_index("s") * 2 + lax.axis_index("c")

        pltpu.sync_copy(ids_hbm.at[:, pl.ds(wid * IBLK, IBLK)], idx_all)

        lane = lax.iota(jnp.int32, 16)
        perms = [(lane + d) & 15 for d in range(16)]
        k8s = [p & 7 for p in perms]
        ptks = [p >> 3 for p in perms]

        def gather_start(j, b):
            pltpu.async_copy(
                table_hbm.at[idx_all.at[j]], rows.at[b], sem_g[b]
            )

        def gather_wait(b):
            pltpu.make_async_copy(
                table_hbm.at[idx_all.at[0]], rows.at[b], sem_g[b]
            ).wait()

        def write_start(j, b):
            pltpu.async_copy(tbuf.at[b], out_hbm.at[j, :, wid], sem_w[b])

        def write_wait(b):
            pltpu.make_async_copy(
                tbuf.at[b], out_hbm.at[0, :, wid], sem_w[b]
            ).wait()

        def transpose_block(b):
            rows_b = rows.at[b]
            tbuf_b = tbuf.at[b]

            @plsc.parallel_loop(0, IBLK // 16, unroll=2)
            def i_body(i0):
                iv = lane + i0 * 16
                for k0 in range(HIDDEN // 16):
                    for d in range(16):
                        v = plsc.load_gather(
                            rows_b, [iv, perms[d] + (k0 * 16)]
                        )
                        plsc.store_scatter(
                            tbuf_b, [ptks[d] + (2 * k0), k8s[d], iv], v
                        )

        def chunk_body(j, b, issue_gather, wait_write):
            gather_wait(b)
            if wait_write:
                write_wait(b)
            transpose_block(b)
            write_start(j, b)
            if issue_gather:
                gather_start(j + NBUF, b)

        for b in range(NBUF):
            gather_start(b, b)

        for b in range(NBUF):
            chunk_body(b, b, True, False)

        def group_body(g, carry):
            j0 = g * NBUF
            for b in range(NBUF):
                chunk_body(j0 + b, b, True, True)
            return carry

        lax.fori_loop(1, NUM_GROUPS - 1, group_body, 0)

        j0 = (NUM_GROUPS - 1) * NBUF
        for b in range(NBUF):
            chunk_body(j0 + b, b, False, True)

        for b in range(NBUF):
            write_wait(b)

    return emb_kernel


_emb = _make_kernel()


@jax.jit
def kernel(phone_ids, table):
    ids_t = phone_ids.T
    phys = _emb(ids_t, table)
    out = phys.transpose(0, 1, 3, 2, 4).reshape(SEQ, HIDDEN, BATCH)
    return out.transpose(2, 0, 1)


# speedup vs baseline: 2.8971x; 1.6188x over previous
import functools

import jax
import jax.numpy as jnp
from jax import lax
from jax.experimental import pallas as pl
from jax.experimental.pallas import tpu as pltpu
from jax.experimental.pallas import tpu_sc as plsc

BATCH = 4096
SEQ = 200
HIDDEN = 64
NUM_WORKERS = 32
IBLK = BATCH // NUM_WORKERS
NBUF = 2
NUM_GROUPS = SEQ // NBUF


def _make_kernel():
    mesh = plsc.VectorSubcoreMesh(core_axis_name="c", subcore_axis_name="s")

    @functools.partial(
        pl.kernel,
        out_type=jax.ShapeDtypeStruct((SEQ, 8, 32, 8, 128), jnp.float32),
        mesh=mesh,
        scratch_types=[
            pltpu.VMEM((SEQ, IBLK), jnp.int32),
            pltpu.VMEM((NBUF, IBLK, HIDDEN), jnp.float32),
            pltpu.VMEM((NBUF, 8, 8, 128), jnp.float32),
            [pltpu.SemaphoreType.DMA] * NBUF,
            [pltpu.SemaphoreType.DMA] * NBUF,
        ],
        compiler_params=pltpu.CompilerParams(
            use_tc_tiling_on_sc=False, needs_layout_passes=False
        ),
    )
    def emb_kernel(ids_hbm, table_hbm, out_hbm, idx_all, rows, tbuf, sem_g, sem_w):
        wid = lax.axis_index("s") * 2 + lax.axis_index("c")

        pltpu.sync_copy(ids_hbm.at[:, pl.ds(wid * IBLK, IBLK)], idx_all)

        lane = lax.iota(jnp.int32, 16)
        perms = [(lane + d) & 15 for d in range(16)]
        k8s = [p & 7 for p in perms]
        ptks = [p >> 3 for p in perms]

        def gather_start(j, b):
            pltpu.async_copy(
                table_hbm.at[idx_all.at[j]], rows.at[b], sem_g[b]
            )

        def gather_wait(b):
            pltpu.make_async_copy(
                table_hbm.at[idx_all.at[0]], rows.at[b], sem_g[b]
            ).wait()

        def write_start(j, b):
            pltpu.async_copy(tbuf.at[b], out_hbm.at[j, :, wid], sem_w[b])

        def write_wait(b):
            pltpu.make_async_copy(
                tbuf.at[b], out_hbm.at[0, :, wid], sem_w[b]
            ).wait()

        def transpose_block(b):
            rows_b = rows.at[b]
            tbuf_b = tbuf.at[b]

            @plsc.parallel_loop(0, IBLK // 16, unroll=4)
            def i_body(i0):
                iv = lane + i0 * 16
                for k0 in range(HIDDEN // 16):
                    for d in range(16):
                        v = plsc.load_gather(
                            rows_b, [iv, perms[d] + (k0 * 16)]
                        )
                        plsc.store_scatter(
                            tbuf_b, [ptks[d] + (2 * k0), k8s[d], iv], v
                        )

        def chunk_body(j, b, issue_gather, wait_write):
            gather_wait(b)
            if wait_write:
                write_wait(b)
            transpose_block(b)
            write_start(j, b)
            if issue_gather:
                gather_start(j + NBUF, b)

        for b in range(NBUF):
            gather_start(b, b)

        for b in range(NBUF):
            chunk_body(b, b, True, False)

        def group_body(g, carry):
            j0 = g * NBUF
            for b in range(NBUF):
                chunk_body(j0 + b, b, True, True)
            return carry

        lax.fori_loop(1, NUM_GROUPS - 1, group_body, 0)

        j0 = (NUM_GROUPS - 1) * NBUF
        for b in range(NBUF):
            chunk_body(j0 + b, b, False, True)

        for b in range(NBUF):
            write_wait(b)

    return emb_kernel


_emb = _make_kernel()


@jax.jit
def kernel(phone_ids, table):
    ids_t = phone_ids.T
    phys = _emb(ids_t, table)
    out = phys.transpose(0, 1, 3, 2, 4).reshape(SEQ, HIDDEN, BATCH)
    return out.transpose(2, 0, 1)
